# baseline (speedup 1.0000x reference)
import jax
import jax.numpy as jnp
from jax import lax
from jax.experimental import pallas as pl
from jax.experimental.pallas import tpu as pltpu

NUM_TOKENS = 32768
D_MODEL = 4096
NUM_EXPERTS = 64

TM = 1024
NBUF = 3


def _outer(x_hbm, wt_ref, oh_hbm, top_hbm, logits_hbm):
    wt = wt_ref[...]

    def inner(x_ref, oh_ref, top_ref, logits_ref):
        logits = jnp.dot(x_ref[...], wt, preferred_element_type=jnp.float32)
        m = jnp.max(logits, axis=1, keepdims=True)
        s = jnp.sum(jnp.exp(logits - m), axis=1, keepdims=True)
        ii = lax.broadcasted_iota(jnp.int32, logits.shape, 1)
        cand = jnp.where(logits == m, ii, NUM_EXPERTS)
        first = jnp.min(cand, axis=1, keepdims=True)
        oh_ref[...] = (ii == first).astype(jnp.int32)
        top_ref[...] = (1.0 / s)[:, 0]
        logits_ref[...] = logits

    pipeline = pltpu.emit_pipeline(
        inner,
        grid=(NUM_TOKENS // TM,),
        in_specs=[
            pl.BlockSpec(
                (TM, D_MODEL),
                lambda i: (i, 0),
                pipeline_mode=pl.Buffered(buffer_count=NBUF, use_lookahead=True),
            ),
        ],
        out_specs=[
            pl.BlockSpec((TM, NUM_EXPERTS), lambda i: (i, 0)),
            pl.BlockSpec((TM,), lambda i: (i,)),
            pl.BlockSpec((TM, NUM_EXPERTS), lambda i: (i, 0)),
        ],
        dimension_semantics=(pltpu.PARALLEL,),
    )
    pipeline(x_hbm, oh_hbm, top_hbm, logits_hbm)


@jax.jit
def kernel(x, W):
    wt = W.T
    oh, top, logits = pl.pallas_call(
        _outer,
        in_specs=[
            pl.BlockSpec(memory_space=pl.ANY),
            pl.BlockSpec((D_MODEL, NUM_EXPERTS), lambda: (0, 0)),
        ],
        out_specs=[
            pl.BlockSpec(memory_space=pl.ANY),
            pl.BlockSpec(memory_space=pl.ANY),
            pl.BlockSpec(memory_space=pl.ANY),
        ],
        out_shape=[
            jax.ShapeDtypeStruct((NUM_TOKENS, NUM_EXPERTS), jnp.int32),
            jax.ShapeDtypeStruct((NUM_TOKENS,), jnp.float32),
            jax.ShapeDtypeStruct((NUM_TOKENS, NUM_EXPERTS), jnp.float32),
        ],
    )(x, wt)
    return oh, top.reshape(NUM_TOKENS, 1), logits


# speedup vs baseline: 1.0036x; 1.0036x over previous
import jax
import jax.numpy as jnp
from jax import lax
from jax.experimental import pallas as pl
from jax.experimental.pallas import tpu as pltpu

NUM_TOKENS = 32768
D_MODEL = 4096
NUM_EXPERTS = 64

TM = 1024
RSPLIT = 4
TR = TM // RSPLIT
NBLK = NUM_TOKENS // TM


def _router_kernel(*refs):
    x_refs = refs[:RSPLIT]
    wt_ref = refs[RSPLIT]
    oh_ref, top_ref, logits_ref = refs[RSPLIT + 1:RSPLIT + 4]
    sc = refs[RSPLIT + 4]

    logits = sc[...]
    m = jnp.max(logits, axis=1, keepdims=True)
    s = jnp.sum(jnp.exp(logits - m), axis=1, keepdims=True)
    ii = lax.broadcasted_iota(jnp.int32, logits.shape, 1)
    cand = jnp.where(logits == m, ii, NUM_EXPERTS)
    first = jnp.min(cand, axis=1, keepdims=True)
    oh_ref[...] = (ii == first).astype(jnp.int32)
    top_ref[...] = (1.0 / s)[:, 0]
    logits_ref[...] = logits

    sc[...] = jnp.concatenate(
        [jnp.dot(xr[...], wt_ref[...], preferred_element_type=jnp.float32)
         for xr in x_refs],
        axis=0,
    )


def _clamp_blk(i):
    return jnp.minimum(i, NBLK - 1)


def _prev_blk(i):
    return jnp.maximum(i - 1, 0)


@jax.jit
def kernel(x, W):
    wt = W.T
    grid = (NBLK + 1,)
    oh, top, logits = pl.pallas_call(
        _router_kernel,
        grid=grid,
        in_specs=[
            pl.BlockSpec(
                (TR, D_MODEL),
                lambda i, r=r: (_clamp_blk(i) * RSPLIT + r, 0),
            )
            for r in range(RSPLIT)
        ]
        + [pl.BlockSpec((D_MODEL, NUM_EXPERTS), lambda i: (0, 0))],
        out_specs=[
            pl.BlockSpec((TM, NUM_EXPERTS), lambda i: (_prev_blk(i), 0)),
            pl.BlockSpec((TM,), lambda i: (_prev_blk(i),)),
            pl.BlockSpec((TM, NUM_EXPERTS), lambda i: (_prev_blk(i), 0)),
        ],
        out_shape=[
            jax.ShapeDtypeStruct((NUM_TOKENS, NUM_EXPERTS), jnp.int32),
            jax.ShapeDtypeStruct((NUM_TOKENS,), jnp.float32),
            jax.ShapeDtypeStruct((NUM_TOKENS, NUM_EXPERTS), jnp.float32),
        ],
        scratch_shapes=[pltpu.VMEM((TM, NUM_EXPERTS), jnp.float32)],
        compiler_params=pltpu.CompilerParams(
            dimension_semantics=("arbitrary",),
        ),
    )(*([x] * RSPLIT + [wt]))
    return oh, top.reshape(NUM_TOKENS, 1), logits
